# Initial kernel scaffold; baseline (speedup 1.0000x reference)
import jax
import jax.numpy as jnp
from jax.experimental import pallas as pl


def kernel(x, idxs):
    raise NotImplementedError("write your pallas kernel here")


# speedup vs baseline: 1.9131x; 1.9131x over previous
import functools

import jax
import jax.numpy as jnp
from jax import lax
from jax.experimental import pallas as pl
from jax.experimental.pallas import tpu as pltpu
from jax.experimental.pallas import tpu_sc as plsc

_NC = 2
_NS = 16
_NW = _NC * _NS
_C = 4
_L = 16


def _make_sc_kernel(n_pad, k_nb, f_feat, n_rows, chunks_per_worker):
    nf = f_feat // _L
    inv_k = 1.0 / float(k_nb)
    mesh = plsc.VectorSubcoreMesh(core_axis_name="c", subcore_axis_name="s")

    @functools.partial(
        pl.kernel,
        mesh=mesh,
        out_type=jax.ShapeDtypeStruct((n_pad, 2 * f_feat), jnp.float32),
        scratch_types=[
            pltpu.VMEM((_C * k_nb,), jnp.int32),
            pltpu.VMEM((_C * k_nb, f_feat), jnp.float32),
            pltpu.VMEM((_C, 2 * f_feat), jnp.float32),
            pltpu.SemaphoreType.DMA,
        ],
    )
    def sc_kernel(x_hbm, idxs_hbm, out_hbm, idx_v, rows_v, out_v, sem):
        wid = lax.axis_index("s") * _NC + lax.axis_index("c")
        worker_base = wid * (chunks_per_worker * _C)

        def chunk_body(ci, carry):
            base = worker_base + ci * _C
            pltpu.sync_copy(idxs_hbm.at[pl.ds(base * k_nb, _C * k_nb)], idx_v)
            pltpu.async_copy(x_hbm.at[idx_v], rows_v, sem).wait()

            for v in range(_C):
                r0 = v * k_nb
                first = [rows_v[r0, pl.ds(f * _L, _L)] for f in range(nf)]
                init = (tuple(first), tuple(first))

                def k_body(k, acc, r0=r0):
                    sums, maxs = acc
                    vals = [rows_v[r0 + k, pl.ds(f * _L, _L)] for f in range(nf)]
                    sums = tuple(s + val for s, val in zip(sums, vals))
                    maxs = tuple(jnp.maximum(m, val) for m, val in zip(maxs, vals))
                    return sums, maxs

                sums, maxs = lax.fori_loop(1, k_nb, k_body, init)
                for f in range(nf):
                    out_v[v, pl.ds(f * _L, _L)] = sums[f] * inv_k
                    out_v[v, pl.ds(f_feat + f * _L, _L)] = maxs[f]

            pltpu.sync_copy(out_v, out_hbm.at[pl.ds(base, _C)])
            return carry

        lax.fori_loop(0, chunks_per_worker, chunk_body, 0)

    return sc_kernel


def kernel(x, idxs):
    n, f_feat = x.shape
    k_nb = idxs.shape[1]
    block = _NW * _C
    n_pad = ((n + block - 1) // block) * block
    chunks_per_worker = n_pad // block

    idxs_flat = jnp.pad(idxs, ((0, n_pad - n), (0, 0))).reshape(-1)
    sc_kernel = _make_sc_kernel(n_pad, k_nb, f_feat, n, chunks_per_worker)
    out = sc_kernel(x, idxs_flat)
    return out[:n]
